# baseline (speedup 1.0000x reference)
import functools

import jax
import jax.numpy as jnp
from jax import lax
from jax.experimental import pallas as pl
from jax.experimental.pallas import tpu as pltpu
from jax.experimental.pallas import tpu_sc as plsc

_N, _E, _D, _H, _G = 10000, 320000, 128, 64, 64
_NC, _NS, _CH = 2, 16, 128
_KCH = 81
_EPAD = _NC * _NS * _KCH * _CH
_NP = 10240
_ACC = _NP
_RPT = _ACC // _NS
_OPT = _RPT
_RING = 3
_BN = 640
_GRID = _NP // _BN

_mesh = plsc.VectorSubcoreMesh(
    core_axis_name="c", subcore_axis_name="s", num_cores=_NC, num_subcores=_NS
)


def _zero_rows(buf, ncols):
    z = jnp.zeros((16,), jnp.float32)

    def row(r, carry):
        for q in range(ncols // 16):
            buf[r, pl.ds(q * 16, 16)] = z
        return carry

    lax.fori_loop(0, _CH, row, 0)


def _zero_acc_slice(zbuf, acc, base):
    off = 0
    while off < _RPT:
        n = min(_CH, _RPT - off)
        pltpu.sync_copy(zbuf.at[pl.ds(0, n)], acc.at[pl.ds(base + off, n)])
        off += n


@functools.partial(
    pl.kernel,
    out_type=jax.ShapeDtypeStruct((_NC, _NP, 16), jnp.float32),
    mesh=_mesh,
    compiler_params=pltpu.CompilerParams(use_tc_tiling_on_sc=False),
    scratch_types=[
        pltpu.VMEM((_KCH, _CH), jnp.int32),
        pltpu.VMEM((_CH, 16), jnp.float32),
        pltpu.VMEM_SHARED((_ACC, 16), jnp.float32),
        pltpu.SemaphoreType.DMA,
    ],
)
def _sc_deg(dst_hbm, out_hbm, didx, obuf, acc, sem):
    c = lax.axis_index("c")
    s = lax.axis_index("s")
    _zero_rows(obuf, 16)
    _zero_acc_slice(obuf, acc, s * _RPT)
    one = jnp.full((16,), 1.0, jnp.float32)

    def row(r, carry):
        obuf[r, :] = one
        return carry

    lax.fori_loop(0, _CH, row, 0)
    plsc.subcore_barrier()
    pltpu.sync_copy(dst_hbm.at[c, s], didx)

    def fire(k, carry):
        pltpu.async_copy(obuf, acc.at[didx.at[k]], sem, add=True)
        return carry

    lax.fori_loop(0, _KCH, fire, 0)

    def drain(k, carry):
        pltpu.make_async_copy(obuf, acc.at[didx.at[0]], sem).wait()
        return carry

    lax.fori_loop(0, _KCH, drain, 0)
    plsc.subcore_barrier()
    pltpu.sync_copy(acc.at[pl.ds(s * _OPT, _OPT)], out_hbm.at[c, pl.ds(s * _OPT, _OPT)])


@functools.partial(
    pl.kernel,
    out_type=jax.ShapeDtypeStruct((_NC, _NP, _H), jnp.float32),
    mesh=_mesh,
    compiler_params=pltpu.CompilerParams(use_tc_tiling_on_sc=False),
    scratch_types=[
        pltpu.VMEM((_KCH, _CH), jnp.int32),
        pltpu.VMEM((_KCH, _CH), jnp.int32),
        [pltpu.VMEM((_CH, _H), jnp.float32) for _ in range(_RING)],
        pltpu.VMEM_SHARED((_ACC, _H), jnp.float32),
        pltpu.VMEM_SHARED((_NP, _H), jnp.float32),
        [pltpu.SemaphoreType.DMA for _ in range(_RING)],
        [pltpu.SemaphoreType.DMA for _ in range(_RING)],
    ],
)
def _sc_agg(v_hbm, src_hbm, dst_hbm, out_hbm, sidx, didx, rows, acc, vs, gsem, ssem):
    c = lax.axis_index("c")
    s = lax.axis_index("s")
    pltpu.async_copy(
        v_hbm.at[pl.ds(s * _RPT, _RPT)], vs.at[pl.ds(s * _RPT, _RPT)], gsem[0]
    )
    _zero_rows(rows[0], _H)
    _zero_acc_slice(rows[0], acc, s * _RPT)
    pltpu.make_async_copy(
        v_hbm.at[pl.ds(s * _RPT, _RPT)], vs.at[pl.ds(s * _RPT, _RPT)], gsem[0]
    ).wait()
    plsc.subcore_barrier()
    pltpu.sync_copy(src_hbm.at[c, s], sidx)
    pltpu.sync_copy(dst_hbm.at[c, s], didx)

    for j in range(_RING):
        pltpu.async_copy(vs.at[sidx.at[j]], rows[j], gsem[j])

    def rnd(r, carry):
        for j in range(_RING):
            k = _RING * r + j
            pltpu.make_async_copy(vs.at[sidx.at[k]], rows[j], gsem[j]).wait()
            pltpu.async_copy(rows[j], acc.at[didx.at[k]], ssem[j], add=True)
        for j in range(_RING):
            nxt = jnp.where(r + 1 < _KCH // _RING, _RING * r + j + _RING, j)
            pltpu.make_async_copy(rows[j], acc.at[didx.at[0]], ssem[j]).wait()
            pltpu.async_copy(vs.at[sidx.at[nxt]], rows[j], gsem[j])
        return carry

    lax.fori_loop(0, _KCH // _RING, rnd, 0)
    for j in range(_RING):
        pltpu.make_async_copy(vs.at[sidx.at[0]], rows[j], gsem[j]).wait()
    plsc.subcore_barrier()
    pltpu.sync_copy(acc.at[pl.ds(s * _OPT, _OPT)], out_hbm.at[c, pl.ds(s * _OPT, _OPT)])


def _tca_body(degp_ref, x_ref, w1_ref, v1_ref, dis_ref):
    d = degp_ref[0] + degp_ref[1]
    dis = lax.rsqrt(d + 1.0)[:, 0:1]
    u = jnp.dot(x_ref[...], w1_ref[...], preferred_element_type=jnp.float32)
    v1_ref[...] = dis * u
    dis_ref[...] = dis


def _tca(degp, x, w1):
    return pl.pallas_call(
        _tca_body,
        grid=(_GRID,),
        in_specs=[
            pl.BlockSpec((_NC, _BN, 16), lambda i: (0, i, 0)),
            pl.BlockSpec((_BN, _D), lambda i: (i, 0)),
            pl.BlockSpec((_D, _H), lambda i: (0, 0)),
        ],
        out_specs=[
            pl.BlockSpec((_BN, _H), lambda i: (i, 0)),
            pl.BlockSpec((_BN, 1), lambda i: (i, 0)),
        ],
        out_shape=[
            jax.ShapeDtypeStruct((_NP, _H), jnp.float32),
            jax.ShapeDtypeStruct((_NP, 1), jnp.float32),
        ],
    )(degp, x, w1)


def _tcb_body(p_ref, v1_ref, dis_ref, b1_ref, w2_ref, v2_ref):
    t = p_ref[0] + p_ref[1] + v1_ref[...]
    dis = dis_ref[...]
    h = jnp.maximum(dis * t + b1_ref[...], 0.0)
    v2_ref[...] = dis * jnp.dot(h, w2_ref[...], preferred_element_type=jnp.float32)


def _tcb(p, v1, dis, b1, w2):
    return pl.pallas_call(
        _tcb_body,
        grid=(_GRID,),
        in_specs=[
            pl.BlockSpec((_NC, _BN, _H), lambda i: (0, i, 0)),
            pl.BlockSpec((_BN, _H), lambda i: (i, 0)),
            pl.BlockSpec((_BN, 1), lambda i: (i, 0)),
            pl.BlockSpec((1, _H), lambda i: (0, 0)),
            pl.BlockSpec((_H, _H), lambda i: (0, 0)),
        ],
        out_specs=pl.BlockSpec((_BN, _H), lambda i: (i, 0)),
        out_shape=jax.ShapeDtypeStruct((_NP, _H), jnp.float32),
    )(p, v1, dis, b1, w2)


def _tcc_body(p_ref, v2_ref, dis_ref, b2_ref, batch_ref, w3_ref, b3_ref,
              out_ref, sums, counts):
    i = pl.program_id(0)

    @pl.when(i == 0)
    def _():
        sums[...] = jnp.zeros_like(sums)
        counts[...] = jnp.zeros_like(counts)

    t = p_ref[0] + p_ref[1] + v2_ref[...]
    h = jnp.maximum(dis_ref[...] * t + b2_ref[...], 0.0)
    gids = lax.broadcasted_iota(jnp.int32, (_BN, 128), 1)
    onehot = (batch_ref[...] == gids).astype(jnp.float32)
    dn = (((0,), (0,)), ((), ()))
    sums[...] += lax.dot_general(onehot, h, dn, preferred_element_type=jnp.float32)
    ones = jnp.ones((_BN, 128), jnp.float32)
    counts[...] += lax.dot_general(onehot, ones, dn, preferred_element_type=jnp.float32)

    @pl.when(i == _GRID - 1)
    def _():
        cnt = counts[:, 0:1]
        pooled = sums[...] / jnp.maximum(cnt, 1.0)
        out_ref[...] = (
            jnp.dot(pooled, w3_ref[...], preferred_element_type=jnp.float32)
            + b3_ref[...]
        )


def _tcc(p, v2, dis, b2, batch2, w3p, b3p):
    return pl.pallas_call(
        _tcc_body,
        grid=(_GRID,),
        in_specs=[
            pl.BlockSpec((_NC, _BN, _H), lambda i: (0, i, 0)),
            pl.BlockSpec((_BN, _H), lambda i: (i, 0)),
            pl.BlockSpec((_BN, 1), lambda i: (i, 0)),
            pl.BlockSpec((1, _H), lambda i: (0, 0)),
            pl.BlockSpec((_BN, 1), lambda i: (i, 0)),
            pl.BlockSpec((_H, 128), lambda i: (0, 0)),
            pl.BlockSpec((1, 128), lambda i: (0, 0)),
        ],
        out_specs=pl.BlockSpec((128, 128), lambda i: (0, 0)),
        out_shape=jax.ShapeDtypeStruct((128, 128), jnp.float32),
        scratch_shapes=[
            pltpu.VMEM((128, _H), jnp.float32),
            pltpu.VMEM((128, 128), jnp.float32),
        ],
    )(p, v2, dis, b2, batch2, w3p, b3p)


def kernel(x, edge_index, batch, W1, b1, W2, b2, W3, b3):
    pad = _EPAD - _E
    srcp = jnp.concatenate(
        [edge_index[0], jnp.zeros((pad,), jnp.int32)]
    ).reshape(_NC, _NS, _KCH, _CH)
    pad_dst = _N + (jnp.arange(pad, dtype=jnp.int32) % (_NP - _N))
    dstp = jnp.concatenate([edge_index[1], pad_dst]).reshape(_NC, _NS, _KCH, _CH)

    xp = jnp.pad(x, ((0, _NP - _N), (0, 0)))
    batchp = jnp.pad(batch, (0, _NP - _N), constant_values=_G)
    degp = _sc_deg(dstp)
    v1, dis = _tca(degp, xp, W1)
    p1 = _sc_agg(v1, srcp, dstp)
    v2 = _tcb(p1, v1, dis, b1.reshape(1, _H), W2)
    p2 = _sc_agg(v2, srcp, dstp)
    w3p = jnp.pad(W3, ((0, 0), (0, 128 - W3.shape[1])))
    b3p = jnp.pad(b3, (0, 128 - b3.shape[0])).reshape(1, 128)
    outp = _tcc(p2, v2, dis, b2.reshape(1, _H), batchp.reshape(_NP, 1), w3p, b3p)
    return outp[:_G, : W3.shape[1]]


# speedup vs baseline: 1.1394x; 1.1394x over previous
import functools

import jax
import jax.numpy as jnp
from jax import lax
from jax.experimental import pallas as pl
from jax.experimental.pallas import tpu as pltpu
from jax.experimental.pallas import tpu_sc as plsc

_N, _E, _D, _H, _G = 10000, 320000, 128, 64, 64
_NC, _NS, _CH = 2, 16, 128
_KCH = 80
_EPAD = _NC * _NS * _KCH * _CH
_NP = 10240
_ACC = _NP
_RPT = _ACC // _NS
_OPT = _RPT
_RING = 2
_BN = 640
_GRID = _NP // _BN

_mesh = plsc.VectorSubcoreMesh(
    core_axis_name="c", subcore_axis_name="s", num_cores=_NC, num_subcores=_NS
)


def _zero_rows(buf, ncols):
    z = jnp.zeros((16,), jnp.float32)

    def row(r, carry):
        for q in range(ncols // 16):
            buf[r, pl.ds(q * 16, 16)] = z
        return carry

    lax.fori_loop(0, _CH, row, 0)


def _zero_acc_slice(zbuf, acc, base):
    off = 0
    while off < _RPT:
        n = min(_CH, _RPT - off)
        pltpu.sync_copy(zbuf.at[pl.ds(0, n)], acc.at[pl.ds(base + off, n)])
        off += n


@functools.partial(
    pl.kernel,
    out_type=jax.ShapeDtypeStruct((_NC, _NP, 16), jnp.float32),
    mesh=_mesh,
    compiler_params=pltpu.CompilerParams(use_tc_tiling_on_sc=False),
    scratch_types=[
        pltpu.VMEM((_KCH, _CH), jnp.int32),
        pltpu.VMEM((_CH, 16), jnp.float32),
        pltpu.VMEM_SHARED((_ACC, 16), jnp.float32),
        pltpu.SemaphoreType.DMA,
    ],
)
def _sc_deg(dst_hbm, out_hbm, didx, obuf, acc, sem):
    c = lax.axis_index("c")
    s = lax.axis_index("s")
    _zero_rows(obuf, 16)
    _zero_acc_slice(obuf, acc, s * _RPT)
    one = jnp.full((16,), 1.0, jnp.float32)

    def row(r, carry):
        obuf[r, :] = one
        return carry

    lax.fori_loop(0, _CH, row, 0)
    plsc.subcore_barrier()
    pltpu.sync_copy(dst_hbm.at[c, s], didx)

    def fire(k, carry):
        pltpu.async_copy(obuf, acc.at[didx.at[k]], sem, add=True)
        return carry

    lax.fori_loop(0, _KCH, fire, 0)

    def drain(k, carry):
        pltpu.make_async_copy(obuf, acc.at[didx.at[0]], sem).wait()
        return carry

    lax.fori_loop(0, _KCH, drain, 0)
    plsc.subcore_barrier()
    pltpu.sync_copy(acc.at[pl.ds(s * _OPT, _OPT)], out_hbm.at[c, pl.ds(s * _OPT, _OPT)])


@functools.partial(
    pl.kernel,
    out_type=jax.ShapeDtypeStruct((_NC, _NP, _H), jnp.float32),
    mesh=_mesh,
    compiler_params=pltpu.CompilerParams(use_tc_tiling_on_sc=False),
    scratch_types=[
        pltpu.VMEM((_KCH, _CH), jnp.int32),
        pltpu.VMEM((_KCH, _CH), jnp.int32),
        [pltpu.VMEM((_CH, _H), jnp.float32) for _ in range(_RING)],
        pltpu.VMEM_SHARED((_ACC, _H), jnp.float32),
        pltpu.VMEM_SHARED((_NP, _H), jnp.float32),
        [pltpu.SemaphoreType.DMA for _ in range(_RING)],
        [pltpu.SemaphoreType.DMA for _ in range(_RING)],
    ],
)
def _sc_agg(v_hbm, src_hbm, dst_hbm, out_hbm, sidx, didx, rows, acc, vs, gsem, ssem):
    c = lax.axis_index("c")
    s = lax.axis_index("s")
    pltpu.async_copy(
        v_hbm.at[pl.ds(s * _RPT, _RPT)], vs.at[pl.ds(s * _RPT, _RPT)], gsem[0]
    )
    _zero_rows(rows[0], _H)
    _zero_acc_slice(rows[0], acc, s * _RPT)
    pltpu.make_async_copy(
        v_hbm.at[pl.ds(s * _RPT, _RPT)], vs.at[pl.ds(s * _RPT, _RPT)], gsem[0]
    ).wait()
    plsc.subcore_barrier()
    pltpu.sync_copy(src_hbm.at[c, s], sidx)
    pltpu.sync_copy(dst_hbm.at[c, s], didx)

    for j in range(_RING):
        pltpu.async_copy(vs.at[sidx.at[j]], rows[j], gsem[j])

    def rnd(r, carry):
        for j in range(_RING):
            k = _RING * r + j
            pltpu.make_async_copy(vs.at[sidx.at[k]], rows[j], gsem[j]).wait()
            pltpu.async_copy(rows[j], acc.at[didx.at[k]], ssem[j], add=True)
        for j in range(_RING):
            nxt = jnp.where(r + 1 < _KCH // _RING, _RING * r + j + _RING, j)
            pltpu.make_async_copy(rows[j], acc.at[didx.at[0]], ssem[j]).wait()
            pltpu.async_copy(vs.at[sidx.at[nxt]], rows[j], gsem[j])
        return carry

    lax.fori_loop(0, _KCH // _RING, rnd, 0)
    for j in range(_RING):
        pltpu.make_async_copy(vs.at[sidx.at[0]], rows[j], gsem[j]).wait()
    plsc.subcore_barrier()
    pltpu.sync_copy(acc.at[pl.ds(s * _OPT, _OPT)], out_hbm.at[c, pl.ds(s * _OPT, _OPT)])


def _tca_body(degp_ref, x_ref, w1_ref, v1_ref, dis_ref):
    d = degp_ref[0] + degp_ref[1]
    dis = lax.rsqrt(d + 1.0)[:, 0:1]
    u = jnp.dot(x_ref[...], w1_ref[...], preferred_element_type=jnp.float32)
    v1_ref[...] = dis * u
    dis_ref[...] = dis


def _tca(degp, x, w1):
    return pl.pallas_call(
        _tca_body,
        grid=(_GRID,),
        in_specs=[
            pl.BlockSpec((_NC, _BN, 16), lambda i: (0, i, 0)),
            pl.BlockSpec((_BN, _D), lambda i: (i, 0)),
            pl.BlockSpec((_D, _H), lambda i: (0, 0)),
        ],
        out_specs=[
            pl.BlockSpec((_BN, _H), lambda i: (i, 0)),
            pl.BlockSpec((_BN, 1), lambda i: (i, 0)),
        ],
        out_shape=[
            jax.ShapeDtypeStruct((_NP, _H), jnp.float32),
            jax.ShapeDtypeStruct((_NP, 1), jnp.float32),
        ],
    )(degp, x, w1)


def _tcb_body(p_ref, v1_ref, dis_ref, b1_ref, w2_ref, v2_ref):
    t = p_ref[0] + p_ref[1] + v1_ref[...]
    dis = dis_ref[...]
    h = jnp.maximum(dis * t + b1_ref[...], 0.0)
    v2_ref[...] = dis * jnp.dot(h, w2_ref[...], preferred_element_type=jnp.float32)


def _tcb(p, v1, dis, b1, w2):
    return pl.pallas_call(
        _tcb_body,
        grid=(_GRID,),
        in_specs=[
            pl.BlockSpec((_NC, _BN, _H), lambda i: (0, i, 0)),
            pl.BlockSpec((_BN, _H), lambda i: (i, 0)),
            pl.BlockSpec((_BN, 1), lambda i: (i, 0)),
            pl.BlockSpec((1, _H), lambda i: (0, 0)),
            pl.BlockSpec((_H, _H), lambda i: (0, 0)),
        ],
        out_specs=pl.BlockSpec((_BN, _H), lambda i: (i, 0)),
        out_shape=jax.ShapeDtypeStruct((_NP, _H), jnp.float32),
    )(p, v1, dis, b1, w2)


def _tcc_body(p_ref, v2_ref, dis_ref, b2_ref, batch_ref, w3_ref, b3_ref,
              out_ref, sums, counts):
    i = pl.program_id(0)

    @pl.when(i == 0)
    def _():
        sums[...] = jnp.zeros_like(sums)
        counts[...] = jnp.zeros_like(counts)

    t = p_ref[0] + p_ref[1] + v2_ref[...]
    h = jnp.maximum(dis_ref[...] * t + b2_ref[...], 0.0)
    gids = lax.broadcasted_iota(jnp.int32, (_BN, 128), 1)
    onehot = (batch_ref[...] == gids).astype(jnp.float32)
    dn = (((0,), (0,)), ((), ()))
    sums[...] += lax.dot_general(onehot, h, dn, preferred_element_type=jnp.float32)
    ones = jnp.ones((_BN, 128), jnp.float32)
    counts[...] += lax.dot_general(onehot, ones, dn, preferred_element_type=jnp.float32)

    @pl.when(i == _GRID - 1)
    def _():
        cnt = counts[:, 0:1]
        pooled = sums[...] / jnp.maximum(cnt, 1.0)
        out_ref[...] = (
            jnp.dot(pooled, w3_ref[...], preferred_element_type=jnp.float32)
            + b3_ref[...]
        )


def _tcc(p, v2, dis, b2, batch2, w3p, b3p):
    return pl.pallas_call(
        _tcc_body,
        grid=(_GRID,),
        in_specs=[
            pl.BlockSpec((_NC, _BN, _H), lambda i: (0, i, 0)),
            pl.BlockSpec((_BN, _H), lambda i: (i, 0)),
            pl.BlockSpec((_BN, 1), lambda i: (i, 0)),
            pl.BlockSpec((1, _H), lambda i: (0, 0)),
            pl.BlockSpec((_BN, 1), lambda i: (i, 0)),
            pl.BlockSpec((_H, 128), lambda i: (0, 0)),
            pl.BlockSpec((1, 128), lambda i: (0, 0)),
        ],
        out_specs=pl.BlockSpec((128, 128), lambda i: (0, 0)),
        out_shape=jax.ShapeDtypeStruct((128, 128), jnp.float32),
        scratch_shapes=[
            pltpu.VMEM((128, _H), jnp.float32),
            pltpu.VMEM((128, 128), jnp.float32),
        ],
    )(p, v2, dis, b2, batch2, w3p, b3p)


def kernel(x, edge_index, batch, W1, b1, W2, b2, W3, b3):
    pad = _EPAD - _E
    srcp = jnp.concatenate(
        [edge_index[0], jnp.zeros((pad,), jnp.int32)]
    ).reshape(_NC, _NS, _KCH, _CH)
    pad_dst = _N + (jnp.arange(pad, dtype=jnp.int32) % (_NP - _N))
    dstp = jnp.concatenate([edge_index[1], pad_dst]).reshape(_NC, _NS, _KCH, _CH)

    xp = jnp.pad(x, ((0, _NP - _N), (0, 0)))
    batchp = jnp.pad(batch, (0, _NP - _N), constant_values=_G)
    degp = _sc_deg(dstp)
    v1, dis = _tca(degp, xp, W1)
    p1 = _sc_agg(v1, srcp, dstp)
    v2 = _tcb(p1, v1, dis, b1.reshape(1, _H), W2)
    p2 = _sc_agg(v2, srcp, dstp)
    w3p = jnp.pad(W3, ((0, 0), (0, 128 - W3.shape[1])))
    b3p = jnp.pad(b3, (0, 128 - b3.shape[0])).reshape(1, 128)
    outp = _tcc(p2, v2, dis, b2.reshape(1, _H), batchp.reshape(_NP, 1), w3p, b3p)
    return outp[:_G, : W3.shape[1]]


# speedup vs baseline: 1.2090x; 1.0611x over previous
import functools

import jax
import jax.numpy as jnp
from jax import lax
from jax.experimental import pallas as pl
from jax.experimental.pallas import tpu as pltpu
from jax.experimental.pallas import tpu_sc as plsc

_N, _E, _D, _H, _G = 10000, 320000, 128, 64, 64
_NC, _NS, _CH = 2, 16, 128
_KCH = 80
_EPAD = _NC * _NS * _KCH * _CH
_NP = 10240
_ACC = _NP
_RPT = _ACC // _NS
_OPT = _RPT
_RING = 2
_BN = 2048
_GRID = _NP // _BN

_mesh = plsc.VectorSubcoreMesh(
    core_axis_name="c", subcore_axis_name="s", num_cores=_NC, num_subcores=_NS
)


def _zero_rows(buf, ncols):
    z = jnp.zeros((16,), jnp.float32)

    def row(r, carry):
        for q in range(ncols // 16):
            buf[r, pl.ds(q * 16, 16)] = z
        return carry

    lax.fori_loop(0, _CH, row, 0)


def _zero_acc_slice(zbuf, acc, base):
    off = 0
    while off < _RPT:
        n = min(_CH, _RPT - off)
        pltpu.sync_copy(zbuf.at[pl.ds(0, n)], acc.at[pl.ds(base + off, n)])
        off += n


@functools.partial(
    pl.kernel,
    out_type=jax.ShapeDtypeStruct((_NC, _NP, 16), jnp.float32),
    mesh=_mesh,
    compiler_params=pltpu.CompilerParams(use_tc_tiling_on_sc=False),
    scratch_types=[
        pltpu.VMEM((_KCH, _CH), jnp.int32),
        pltpu.VMEM((_CH, 16), jnp.float32),
        pltpu.VMEM_SHARED((_ACC, 16), jnp.float32),
        pltpu.SemaphoreType.DMA,
    ],
)
def _sc_deg(dst_hbm, out_hbm, didx, obuf, acc, sem):
    c = lax.axis_index("c")
    s = lax.axis_index("s")
    _zero_rows(obuf, 16)
    _zero_acc_slice(obuf, acc, s * _RPT)
    one = jnp.full((16,), 1.0, jnp.float32)

    def row(r, carry):
        obuf[r, :] = one
        return carry

    lax.fori_loop(0, _CH, row, 0)
    plsc.subcore_barrier()
    pltpu.sync_copy(dst_hbm.at[c, s], didx)

    def fire(k, carry):
        pltpu.async_copy(obuf, acc.at[didx.at[k]], sem, add=True)
        return carry

    lax.fori_loop(0, _KCH, fire, 0)

    def drain(k, carry):
        pltpu.make_async_copy(obuf, acc.at[didx.at[0]], sem).wait()
        return carry

    lax.fori_loop(0, _KCH, drain, 0)
    plsc.subcore_barrier()
    pltpu.sync_copy(acc.at[pl.ds(s * _OPT, _OPT)], out_hbm.at[c, pl.ds(s * _OPT, _OPT)])


@functools.partial(
    pl.kernel,
    out_type=jax.ShapeDtypeStruct((_NC, _NP, _H), jnp.float32),
    mesh=_mesh,
    compiler_params=pltpu.CompilerParams(use_tc_tiling_on_sc=False),
    scratch_types=[
        pltpu.VMEM((_KCH, _CH), jnp.int32),
        pltpu.VMEM((_KCH, _CH), jnp.int32),
        [pltpu.VMEM((_CH, _H), jnp.float32) for _ in range(_RING)],
        pltpu.VMEM_SHARED((_ACC, _H), jnp.float32),
        pltpu.VMEM_SHARED((_NP, _H), jnp.float32),
        [pltpu.SemaphoreType.DMA for _ in range(_RING)],
        [pltpu.SemaphoreType.DMA for _ in range(_RING)],
    ],
)
def _sc_agg(v_hbm, src_hbm, dst_hbm, out_hbm, sidx, didx, rows, acc, vs, gsem, ssem):
    c = lax.axis_index("c")
    s = lax.axis_index("s")
    pltpu.async_copy(
        v_hbm.at[pl.ds(s * _RPT, _RPT)], vs.at[pl.ds(s * _RPT, _RPT)], gsem[0]
    )
    _zero_rows(rows[0], _H)
    _zero_acc_slice(rows[0], acc, s * _RPT)
    pltpu.make_async_copy(
        v_hbm.at[pl.ds(s * _RPT, _RPT)], vs.at[pl.ds(s * _RPT, _RPT)], gsem[0]
    ).wait()
    plsc.subcore_barrier()
    pltpu.sync_copy(src_hbm.at[c, s], sidx)
    pltpu.sync_copy(dst_hbm.at[c, s], didx)

    for j in range(_RING):
        pltpu.async_copy(vs.at[sidx.at[j]], rows[j], gsem[j])

    def rnd(r, carry):
        for j in range(_RING):
            k = _RING * r + j
            pltpu.make_async_copy(vs.at[sidx.at[k]], rows[j], gsem[j]).wait()
            pltpu.async_copy(rows[j], acc.at[didx.at[k]], ssem[j], add=True)
        for j in range(_RING):
            nxt = jnp.where(r + 1 < _KCH // _RING, _RING * r + j + _RING, j)
            pltpu.make_async_copy(rows[j], acc.at[didx.at[0]], ssem[j]).wait()
            pltpu.async_copy(vs.at[sidx.at[nxt]], rows[j], gsem[j])
        return carry

    lax.fori_loop(0, _KCH // _RING, rnd, 0)
    for j in range(_RING):
        pltpu.make_async_copy(vs.at[sidx.at[0]], rows[j], gsem[j]).wait()
    plsc.subcore_barrier()
    pltpu.sync_copy(acc.at[pl.ds(s * _OPT, _OPT)], out_hbm.at[c, pl.ds(s * _OPT, _OPT)])


def _tca_body(degp_ref, x_ref, w1_ref, v1_ref, dis_ref):
    d = degp_ref[0] + degp_ref[1]
    dis = lax.rsqrt(d + 1.0)[:, 0:1]
    u = jnp.dot(x_ref[...], w1_ref[...], preferred_element_type=jnp.float32)
    v1_ref[...] = dis * u
    dis_ref[...] = dis


def _tca(degp, x, w1):
    return pl.pallas_call(
        _tca_body,
        grid=(_GRID,),
        in_specs=[
            pl.BlockSpec((_NC, _BN, 16), lambda i: (0, i, 0)),
            pl.BlockSpec((_BN, _D), lambda i: (i, 0)),
            pl.BlockSpec((_D, _H), lambda i: (0, 0)),
        ],
        out_specs=[
            pl.BlockSpec((_BN, _H), lambda i: (i, 0)),
            pl.BlockSpec((_BN, 1), lambda i: (i, 0)),
        ],
        out_shape=[
            jax.ShapeDtypeStruct((_NP, _H), jnp.float32),
            jax.ShapeDtypeStruct((_NP, 1), jnp.float32),
        ],
    )(degp, x, w1)


def _tcb_body(p_ref, v1_ref, dis_ref, b1_ref, w2_ref, v2_ref):
    t = p_ref[0] + p_ref[1] + v1_ref[...]
    dis = dis_ref[...]
    h = jnp.maximum(dis * t + b1_ref[...], 0.0)
    v2_ref[...] = dis * jnp.dot(h, w2_ref[...], preferred_element_type=jnp.float32)


def _tcb(p, v1, dis, b1, w2):
    return pl.pallas_call(
        _tcb_body,
        grid=(_GRID,),
        in_specs=[
            pl.BlockSpec((_NC, _BN, _H), lambda i: (0, i, 0)),
            pl.BlockSpec((_BN, _H), lambda i: (i, 0)),
            pl.BlockSpec((_BN, 1), lambda i: (i, 0)),
            pl.BlockSpec((1, _H), lambda i: (0, 0)),
            pl.BlockSpec((_H, _H), lambda i: (0, 0)),
        ],
        out_specs=pl.BlockSpec((_BN, _H), lambda i: (i, 0)),
        out_shape=jax.ShapeDtypeStruct((_NP, _H), jnp.float32),
    )(p, v1, dis, b1, w2)


def _tcc_body(p_ref, v2_ref, dis_ref, b2_ref, batch_ref, w3_ref, b3_ref,
              out_ref, sums, counts):
    i = pl.program_id(0)

    @pl.when(i == 0)
    def _():
        sums[...] = jnp.zeros_like(sums)
        counts[...] = jnp.zeros_like(counts)

    t = p_ref[0] + p_ref[1] + v2_ref[...]
    h = jnp.maximum(dis_ref[...] * t + b2_ref[...], 0.0)
    gids = lax.broadcasted_iota(jnp.int32, (_BN, 128), 1)
    onehot = (batch_ref[...] == gids).astype(jnp.float32)
    dn = (((0,), (0,)), ((), ()))
    sums[...] += lax.dot_general(onehot, h, dn, preferred_element_type=jnp.float32)
    ones = jnp.ones((_BN, 128), jnp.float32)
    counts[...] += lax.dot_general(onehot, ones, dn, preferred_element_type=jnp.float32)

    @pl.when(i == _GRID - 1)
    def _():
        cnt = counts[:, 0:1]
        pooled = sums[...] / jnp.maximum(cnt, 1.0)
        out_ref[...] = (
            jnp.dot(pooled, w3_ref[...], preferred_element_type=jnp.float32)
            + b3_ref[...]
        )


def _tcc(p, v2, dis, b2, batch2, w3p, b3p):
    return pl.pallas_call(
        _tcc_body,
        grid=(_GRID,),
        in_specs=[
            pl.BlockSpec((_NC, _BN, _H), lambda i: (0, i, 0)),
            pl.BlockSpec((_BN, _H), lambda i: (i, 0)),
            pl.BlockSpec((_BN, 1), lambda i: (i, 0)),
            pl.BlockSpec((1, _H), lambda i: (0, 0)),
            pl.BlockSpec((_BN, 1), lambda i: (i, 0)),
            pl.BlockSpec((_H, 128), lambda i: (0, 0)),
            pl.BlockSpec((1, 128), lambda i: (0, 0)),
        ],
        out_specs=pl.BlockSpec((128, 128), lambda i: (0, 0)),
        out_shape=jax.ShapeDtypeStruct((128, 128), jnp.float32),
        scratch_shapes=[
            pltpu.VMEM((128, _H), jnp.float32),
            pltpu.VMEM((128, 128), jnp.float32),
        ],
    )(p, v2, dis, b2, batch2, w3p, b3p)


def kernel(x, edge_index, batch, W1, b1, W2, b2, W3, b3):
    pad = _EPAD - _E
    srcp = jnp.concatenate(
        [edge_index[0], jnp.zeros((pad,), jnp.int32)]
    ).reshape(_NC, _NS, _KCH, _CH)
    pad_dst = _N + (jnp.arange(pad, dtype=jnp.int32) % (_NP - _N))
    dstp = jnp.concatenate([edge_index[1], pad_dst]).reshape(_NC, _NS, _KCH, _CH)

    xp = jnp.pad(x, ((0, _NP - _N), (0, 0)))
    batchp = jnp.pad(batch, (0, _NP - _N), constant_values=_G)
    degp = _sc_deg(dstp)
    v1, dis = _tca(degp, xp, W1)
    p1 = _sc_agg(v1, srcp, dstp)
    v2 = _tcb(p1, v1, dis, b1.reshape(1, _H), W2)
    p2 = _sc_agg(v2, srcp, dstp)
    w3p = jnp.pad(W3, ((0, 0), (0, 128 - W3.shape[1])))
    b3p = jnp.pad(b3, (0, 128 - b3.shape[0])).reshape(1, 128)
    outp = _tcc(p2, v2, dis, b2.reshape(1, _H), batchp.reshape(_NP, 1), w3p, b3p)
    return outp[:_G, : W3.shape[1]]


# speedup vs baseline: 1.2096x; 1.0004x over previous
import functools

import jax
import jax.numpy as jnp
from jax import lax
from jax.experimental import pallas as pl
from jax.experimental.pallas import tpu as pltpu
from jax.experimental.pallas import tpu_sc as plsc

_N, _E, _D, _H, _G = 10000, 320000, 128, 64, 64
_NC, _NS, _CH = 2, 16, 128
_KCH = 80
_EPAD = _NC * _NS * _KCH * _CH
_NP = 10240
_ACC = _NP
_RPT = _ACC // _NS
_OPT = _RPT
_RING = 2
_BN = 2048
_GRID = _NP // _BN

_mesh = plsc.VectorSubcoreMesh(
    core_axis_name="c", subcore_axis_name="s", num_cores=_NC, num_subcores=_NS
)


def _zero_rows(buf, ncols):
    z = jnp.zeros((16,), jnp.float32)

    def row(r, carry):
        for q in range(ncols // 16):
            buf[r, pl.ds(q * 16, 16)] = z
        return carry

    lax.fori_loop(0, _CH, row, 0)


def _zero_acc_slice(zbuf, acc, base, sem):
    spans = []
    off = 0
    while off < _RPT:
        n = min(_CH, _RPT - off)
        spans.append((off, n))
        off += n
    for off, n in spans:
        pltpu.async_copy(zbuf.at[pl.ds(0, n)], acc.at[pl.ds(base + off, n)], sem)
    for off, n in spans:
        pltpu.make_async_copy(zbuf.at[pl.ds(0, n)], acc.at[pl.ds(base + off, n)], sem).wait()


@functools.partial(
    pl.kernel,
    out_type=jax.ShapeDtypeStruct((_NC, _NP, 16), jnp.float32),
    mesh=_mesh,
    compiler_params=pltpu.CompilerParams(use_tc_tiling_on_sc=False),
    scratch_types=[
        pltpu.VMEM((_KCH, _CH), jnp.int32),
        pltpu.VMEM((_CH, 16), jnp.float32),
        pltpu.VMEM_SHARED((_ACC, 16), jnp.float32),
        pltpu.SemaphoreType.DMA,
    ],
)
def _sc_deg(dst_hbm, out_hbm, didx, obuf, acc, sem):
    c = lax.axis_index("c")
    s = lax.axis_index("s")
    _zero_rows(obuf, 16)
    _zero_acc_slice(obuf, acc, s * _RPT, sem)
    one = jnp.full((16,), 1.0, jnp.float32)

    def row(r, carry):
        obuf[r, :] = one
        return carry

    lax.fori_loop(0, _CH, row, 0)
    plsc.subcore_barrier()
    pltpu.sync_copy(dst_hbm.at[c, s], didx)

    def fire(k, carry):
        pltpu.async_copy(obuf, acc.at[didx.at[k]], sem, add=True)
        return carry

    lax.fori_loop(0, _KCH, fire, 0)

    def drain(k, carry):
        pltpu.make_async_copy(obuf, acc.at[didx.at[0]], sem).wait()
        return carry

    lax.fori_loop(0, _KCH, drain, 0)
    plsc.subcore_barrier()
    pltpu.sync_copy(acc.at[pl.ds(s * _OPT, _OPT)], out_hbm.at[c, pl.ds(s * _OPT, _OPT)])


@functools.partial(
    pl.kernel,
    out_type=jax.ShapeDtypeStruct((_NC, _NP, _H), jnp.float32),
    mesh=_mesh,
    compiler_params=pltpu.CompilerParams(use_tc_tiling_on_sc=False),
    scratch_types=[
        pltpu.VMEM((_KCH, _CH), jnp.int32),
        pltpu.VMEM((_KCH, _CH), jnp.int32),
        [pltpu.VMEM((_CH, _H), jnp.float32) for _ in range(_RING)],
        pltpu.VMEM_SHARED((_ACC, _H), jnp.float32),
        pltpu.VMEM_SHARED((_NP, _H), jnp.float32),
        [pltpu.SemaphoreType.DMA for _ in range(_RING)],
        [pltpu.SemaphoreType.DMA for _ in range(_RING)],
    ],
)
def _sc_agg(v_hbm, src_hbm, dst_hbm, out_hbm, sidx, didx, rows, acc, vs, gsem, ssem):
    c = lax.axis_index("c")
    s = lax.axis_index("s")
    pltpu.async_copy(
        v_hbm.at[pl.ds(s * _RPT, _RPT)], vs.at[pl.ds(s * _RPT, _RPT)], gsem[0]
    )
    _zero_rows(rows[0], _H)
    _zero_acc_slice(rows[0], acc, s * _RPT, ssem[0])
    pltpu.make_async_copy(
        v_hbm.at[pl.ds(s * _RPT, _RPT)], vs.at[pl.ds(s * _RPT, _RPT)], gsem[0]
    ).wait()
    plsc.subcore_barrier()
    pltpu.sync_copy(src_hbm.at[c, s], sidx)
    pltpu.sync_copy(dst_hbm.at[c, s], didx)

    for j in range(_RING):
        pltpu.async_copy(vs.at[sidx.at[j]], rows[j], gsem[j])

    def rnd(r, carry):
        for j in range(_RING):
            k = _RING * r + j
            pltpu.make_async_copy(vs.at[sidx.at[k]], rows[j], gsem[j]).wait()
            pltpu.async_copy(rows[j], acc.at[didx.at[k]], ssem[j], add=True)
        for j in range(_RING):
            nxt = jnp.where(r + 1 < _KCH // _RING, _RING * r + j + _RING, j)
            pltpu.make_async_copy(rows[j], acc.at[didx.at[0]], ssem[j]).wait()
            pltpu.async_copy(vs.at[sidx.at[nxt]], rows[j], gsem[j])
        return carry

    lax.fori_loop(0, _KCH // _RING, rnd, 0)
    for j in range(_RING):
        pltpu.make_async_copy(vs.at[sidx.at[0]], rows[j], gsem[j]).wait()
    plsc.subcore_barrier()
    pltpu.sync_copy(acc.at[pl.ds(s * _OPT, _OPT)], out_hbm.at[c, pl.ds(s * _OPT, _OPT)])


def _tca_body(degp_ref, x_ref, w1_ref, v1_ref, dis_ref):
    d = degp_ref[0] + degp_ref[1]
    dis = lax.rsqrt(d + 1.0)[:, 0:1]
    u = jnp.dot(x_ref[...], w1_ref[...], preferred_element_type=jnp.float32)
    v1_ref[...] = dis * u
    dis_ref[...] = dis


def _tca(degp, x, w1):
    return pl.pallas_call(
        _tca_body,
        grid=(_GRID,),
        in_specs=[
            pl.BlockSpec((_NC, _BN, 16), lambda i: (0, i, 0)),
            pl.BlockSpec((_BN, _D), lambda i: (i, 0)),
            pl.BlockSpec((_D, _H), lambda i: (0, 0)),
        ],
        out_specs=[
            pl.BlockSpec((_BN, _H), lambda i: (i, 0)),
            pl.BlockSpec((_BN, 1), lambda i: (i, 0)),
        ],
        out_shape=[
            jax.ShapeDtypeStruct((_NP, _H), jnp.float32),
            jax.ShapeDtypeStruct((_NP, 1), jnp.float32),
        ],
    )(degp, x, w1)


def _tcb_body(p_ref, v1_ref, dis_ref, b1_ref, w2_ref, v2_ref):
    t = p_ref[0] + p_ref[1] + v1_ref[...]
    dis = dis_ref[...]
    h = jnp.maximum(dis * t + b1_ref[...], 0.0)
    v2_ref[...] = dis * jnp.dot(h, w2_ref[...], preferred_element_type=jnp.float32)


def _tcb(p, v1, dis, b1, w2):
    return pl.pallas_call(
        _tcb_body,
        grid=(_GRID,),
        in_specs=[
            pl.BlockSpec((_NC, _BN, _H), lambda i: (0, i, 0)),
            pl.BlockSpec((_BN, _H), lambda i: (i, 0)),
            pl.BlockSpec((_BN, 1), lambda i: (i, 0)),
            pl.BlockSpec((1, _H), lambda i: (0, 0)),
            pl.BlockSpec((_H, _H), lambda i: (0, 0)),
        ],
        out_specs=pl.BlockSpec((_BN, _H), lambda i: (i, 0)),
        out_shape=jax.ShapeDtypeStruct((_NP, _H), jnp.float32),
    )(p, v1, dis, b1, w2)


def _tcc_body(p_ref, v2_ref, dis_ref, b2_ref, batch_ref, w3_ref, b3_ref,
              out_ref, sums, counts):
    i = pl.program_id(0)

    @pl.when(i == 0)
    def _():
        sums[...] = jnp.zeros_like(sums)
        counts[...] = jnp.zeros_like(counts)

    t = p_ref[0] + p_ref[1] + v2_ref[...]
    h = jnp.maximum(dis_ref[...] * t + b2_ref[...], 0.0)
    gids = lax.broadcasted_iota(jnp.int32, (_BN, 128), 1)
    onehot = (batch_ref[...] == gids).astype(jnp.float32)
    dn = (((0,), (0,)), ((), ()))
    sums[...] += lax.dot_general(onehot, h, dn, preferred_element_type=jnp.float32)
    ones = jnp.ones((_BN, 128), jnp.float32)
    counts[...] += lax.dot_general(onehot, ones, dn, preferred_element_type=jnp.float32)

    @pl.when(i == _GRID - 1)
    def _():
        cnt = counts[:, 0:1]
        pooled = sums[...] / jnp.maximum(cnt, 1.0)
        out_ref[...] = (
            jnp.dot(pooled, w3_ref[...], preferred_element_type=jnp.float32)
            + b3_ref[...]
        )


def _tcc(p, v2, dis, b2, batch2, w3p, b3p):
    return pl.pallas_call(
        _tcc_body,
        grid=(_GRID,),
        in_specs=[
            pl.BlockSpec((_NC, _BN, _H), lambda i: (0, i, 0)),
            pl.BlockSpec((_BN, _H), lambda i: (i, 0)),
            pl.BlockSpec((_BN, 1), lambda i: (i, 0)),
            pl.BlockSpec((1, _H), lambda i: (0, 0)),
            pl.BlockSpec((_BN, 1), lambda i: (i, 0)),
            pl.BlockSpec((_H, 128), lambda i: (0, 0)),
            pl.BlockSpec((1, 128), lambda i: (0, 0)),
        ],
        out_specs=pl.BlockSpec((128, 128), lambda i: (0, 0)),
        out_shape=jax.ShapeDtypeStruct((128, 128), jnp.float32),
        scratch_shapes=[
            pltpu.VMEM((128, _H), jnp.float32),
            pltpu.VMEM((128, 128), jnp.float32),
        ],
    )(p, v2, dis, b2, batch2, w3p, b3p)


def kernel(x, edge_index, batch, W1, b1, W2, b2, W3, b3):
    pad = _EPAD - _E
    srcp = jnp.concatenate(
        [edge_index[0], jnp.zeros((pad,), jnp.int32)]
    ).reshape(_NC, _NS, _KCH, _CH)
    pad_dst = _N + (jnp.arange(pad, dtype=jnp.int32) % (_NP - _N))
    dstp = jnp.concatenate([edge_index[1], pad_dst]).reshape(_NC, _NS, _KCH, _CH)

    xp = jnp.pad(x, ((0, _NP - _N), (0, 0)))
    batchp = jnp.pad(batch, (0, _NP - _N), constant_values=_G)
    degp = _sc_deg(dstp)
    v1, dis = _tca(degp, xp, W1)
    p1 = _sc_agg(v1, srcp, dstp)
    v2 = _tcb(p1, v1, dis, b1.reshape(1, _H), W2)
    p2 = _sc_agg(v2, srcp, dstp)
    w3p = jnp.pad(W3, ((0, 0), (0, 128 - W3.shape[1])))
    b3p = jnp.pad(b3, (0, 128 - b3.shape[0])).reshape(1, 128)
    outp = _tcc(p2, v2, dis, b2.reshape(1, _H), batchp.reshape(_NP, 1), w3p, b3p)
    return outp[:_G, : W3.shape[1]]
